# Initial kernel scaffold; baseline (speedup 1.0000x reference)
import jax
import jax.numpy as jnp
from jax.experimental import pallas as pl


def kernel(x, gn_w, gn_b, W_qkv, bn_qkv_g, bn_qkv_b, W_proj, bn_proj_g, bn_proj_b, Wg1, bg1, Wg2, bg2):
    raise NotImplementedError("write your pallas kernel here")


# speedup vs baseline: 87.0116x; 87.0116x over previous
import jax
import jax.numpy as jnp
from jax.experimental import pallas as pl
from jax.experimental.pallas import tpu as pltpu

_DIM = 384
_QK = 32
_PD = 96
_N = 1024
_B = 8
_EPS = 1e-5
_SCALE = _QK ** (-0.5)
_HI = jax.lax.Precision.HIGHEST


def _gate_body(x_ref, w1t_ref, b1_ref, w2_ref, b2_ref, out_ref, acc_ref):
    b = pl.program_id(0)

    @pl.when(b == 0)
    def _init():
        acc_ref[0] = jnp.float32(0.0)

    xb = x_ref[0]
    g1 = jnp.dot(xb, w1t_ref[...], preferred_element_type=jnp.float32,
                 precision=_HI)
    g1 = jnp.maximum(g1 + b1_ref[...], 0.0)
    z = jnp.dot(g1, w2_ref[...], preferred_element_type=jnp.float32,
                precision=_HI) + b2_ref[0]
    acc_ref[0] = acc_ref[0] + jnp.sum(jax.nn.sigmoid(z))

    @pl.when(b == _B - 1)
    def _fin():
        gm = acc_ref[0] / jnp.float32(_B * _N)
        gm = jnp.where(jnp.isnan(gm), jnp.float32(0.5), gm)
        out_ref[0] = jnp.clip(
            jnp.floor(jnp.float32(_N) * gm).astype(jnp.int32), 1, _N)


def _main_body(kd_ref, x1_ref, x2_ref, gnw_ref, gnb_ref,
               wq_ref, bq_ref, wk_ref, bk_ref, wv_ref, bv_ref,
               wp1_ref, wp2_ref, bp_ref, out_ref):
    x1 = x1_ref[0]
    x2 = x2_ref[0]

    mu = jnp.mean(x1)
    var = jnp.mean((x1 - mu) ** 2)
    xn = (x1 - mu) * jax.lax.rsqrt(var + _EPS)
    xn = xn * gnw_ref[...] + gnb_ref[...]

    q = jnp.dot(xn, wq_ref[...], preferred_element_type=jnp.float32,
                precision=_HI) + bq_ref[...]
    k = jnp.dot(xn, wk_ref[...], preferred_element_type=jnp.float32,
                precision=_HI) + bk_ref[...]
    v = jnp.dot(xn, wv_ref[...], preferred_element_type=jnp.float32,
                precision=_HI) + bv_ref[...]

    attn = jax.lax.dot_general(
        q, k, (((1,), (1,)), ((), ())),
        preferred_element_type=jnp.float32,
        precision=_HI) * jnp.float32(_SCALE)

    kd = kd_ref[0]

    u = jax.lax.bitcast_convert_type(attn, jnp.uint32)
    uk = jnp.where(u >= jnp.uint32(0x80000000), ~u,
                   u | jnp.uint32(0x80000000))

    def body(i, prefix):
        bit = (31 - i).astype(jnp.uint32)
        cand = prefix | (jnp.uint32(1) << bit)
        cnt = jnp.sum((uk >= cand).astype(jnp.int32), axis=1,
                      keepdims=True)
        return jnp.where(cnt >= kd, cand, prefix)

    theta = jax.lax.fori_loop(0, 32, body,
                              jnp.zeros((_N, 1), jnp.uint32))
    maskf = (uk >= theta).astype(jnp.float32)

    m = jnp.max(attn, axis=1, keepdims=True)
    e = jnp.exp(attn - m) * maskf
    p = e / jnp.sum(e, axis=1, keepdims=True)

    o1 = jnp.dot(p, v, preferred_element_type=jnp.float32,
                 precision=_HI)
    s1 = o1 * jax.nn.sigmoid(o1)
    s2 = x2 * jax.nn.sigmoid(x2)
    y = (jnp.dot(s1, wp1_ref[...], preferred_element_type=jnp.float32,
                 precision=_HI)
         + jnp.dot(s2, wp2_ref[...], preferred_element_type=jnp.float32,
                   precision=_HI)
         + bp_ref[...])
    out_ref[0] = y


def kernel(x, gn_w, gn_b, W_qkv, bn_qkv_g, bn_qkv_b, W_proj, bn_proj_g,
           bn_proj_b, Wg1, bg1, Wg2, bg2):
    Bs, C, Hh, Ww = x.shape
    N = Hh * Ww

    xt = jnp.transpose(x.reshape(Bs, C, N), (0, 2, 1))
    x1t = xt[:, :, :_PD]
    x2t = xt[:, :, _PD:]

    bnq_s = bn_qkv_g / jnp.sqrt(1.0 + _EPS)
    Wqkv_eff = W_qkv * bnq_s[:, None]
    WqT = Wqkv_eff[:_QK].T
    WkT = Wqkv_eff[_QK:2 * _QK].T
    WvT = Wqkv_eff[2 * _QK:].T
    bq = bn_qkv_b[None, :_QK]
    bk = bn_qkv_b[None, _QK:2 * _QK]
    bv = bn_qkv_b[None, 2 * _QK:]

    bnp_s = bn_proj_g / jnp.sqrt(1.0 + _EPS)
    Wproj_eff = (W_proj * bnp_s[:, None]).T
    Wp1 = Wproj_eff[:_PD]
    Wp2 = Wproj_eff[_PD:]
    bp = bn_proj_b[None, :]

    Wg1T = Wg1.T
    bg1r = bg1[None, :]
    Wg2T = Wg2.T

    kd = pl.pallas_call(
        _gate_body,
        grid=(Bs,),
        in_specs=[
            pl.BlockSpec((1, N, C), lambda b: (b, 0, 0)),
            pl.BlockSpec((C, C // 2), lambda b: (0, 0)),
            pl.BlockSpec((1, C // 2), lambda b: (0, 0)),
            pl.BlockSpec((C // 2, 1), lambda b: (0, 0)),
            pl.BlockSpec(memory_space=pltpu.SMEM),
        ],
        out_specs=pl.BlockSpec(memory_space=pltpu.SMEM),
        out_shape=jax.ShapeDtypeStruct((1,), jnp.int32),
        scratch_shapes=[pltpu.SMEM((1,), jnp.float32)],
    )(xt, Wg1T, bg1r, Wg2T, bg2)

    grid_spec = pltpu.PrefetchScalarGridSpec(
        num_scalar_prefetch=1,
        grid=(Bs,),
        in_specs=[
            pl.BlockSpec((1, N, _PD), lambda b, kd: (b, 0, 0)),
            pl.BlockSpec((1, N, C - _PD), lambda b, kd: (b, 0, 0)),
            pl.BlockSpec((1, _PD), lambda b, kd: (0, 0)),
            pl.BlockSpec((1, _PD), lambda b, kd: (0, 0)),
            pl.BlockSpec((_PD, _QK), lambda b, kd: (0, 0)),
            pl.BlockSpec((1, _QK), lambda b, kd: (0, 0)),
            pl.BlockSpec((_PD, _QK), lambda b, kd: (0, 0)),
            pl.BlockSpec((1, _QK), lambda b, kd: (0, 0)),
            pl.BlockSpec((_PD, _PD), lambda b, kd: (0, 0)),
            pl.BlockSpec((1, _PD), lambda b, kd: (0, 0)),
            pl.BlockSpec((_PD, C), lambda b, kd: (0, 0)),
            pl.BlockSpec((C - _PD, C), lambda b, kd: (0, 0)),
            pl.BlockSpec((1, C), lambda b, kd: (0, 0)),
        ],
        out_specs=pl.BlockSpec((1, N, C), lambda b, kd: (b, 0, 0)),
    )
    yt = pl.pallas_call(
        _main_body,
        grid_spec=grid_spec,
        out_shape=jax.ShapeDtypeStruct((Bs, N, C), jnp.float32),
    )(kd, x1t, x2t, gn_w[None, :], gn_b[None, :],
      WqT, bq, WkT, bk, WvT, bv, Wp1, Wp2, bp)

    return jnp.transpose(yt, (0, 2, 1)).reshape(Bs, C, Hh, Ww)
